# Initial kernel scaffold; baseline (speedup 1.0000x reference)
import jax
import jax.numpy as jnp
from jax.experimental import pallas as pl


def kernel(x, edge_index, target_indices, W_self1, W_neigh1, b1, W_self2, W_neigh2, b2):
    raise NotImplementedError("write your pallas kernel here")


# speedup vs baseline: 2.8785x; 2.8785x over previous
import functools

import jax
import jax.numpy as jnp
from jax import lax
from jax.experimental import pallas as pl
from jax.experimental.pallas import tpu as pltpu
from jax.experimental.pallas import tpu_sc as plsc

N_NODES = 10000
N_EDGES = 320000
D = 128
NT = 1024

NC, NS = 2, 16
NW = NC * NS
EPW = N_EDGES // NW
EK = 40
NCHUNK = EPW // EK
SPR = 640
SPR_LAST = N_NODES - (NS - 1) * SPR
ZR = 80
TPS = NT // NS
DEGP_R = 80


def _mesh():
    return plsc.VectorSubcoreMesh(core_axis_name="c", subcore_axis_name="s",
                                  num_cores=NC, num_subcores=NS)


def _per_stripe(sid, fn):
    @pl.when(sid < NS - 1)
    def _():
        fn(pl.multiple_of(sid * SPR, 8), SPR)

    @pl.when(sid == NS - 1)
    def _():
        fn((NS - 1) * SPR, SPR_LAST)


def _fill_rows(buf, nrows, ncol16, val16):
    def frow(i, c):
        for j in range(ncol16):
            buf[i, pl.ds(j * 16, 16)] = val16
        return c

    lax.fori_loop(0, nrows, frow, 0)


def _fill_iota(idx_ref, n, base):
    for j in range(n // 16):
        idx_ref[pl.ds(j * 16, 16)] = lax.iota(jnp.int32, 16) + (base + j * 16)


_IOTA = None


def _deg_groups():
    return [(0, 0), (16, 0), (EK - 16, 16 - (EK - 32))]


def _sc_layer1_body(src_ref, dst_ref, x_ref, eye_ref,
                    agg, degp_out,
                    src_v, dst_v, rows_v, hot_v, prow_v, pcol_v, zbuf, zidx_v,
                    s_agg, s_degp, sem):
    cid = lax.axis_index("c")
    sid = lax.axis_index("s")
    wid = sid * NC + cid

    z16 = jnp.zeros((16,), jnp.float32)
    _fill_rows(zbuf, ZR, D // 16, z16)

    def _zero_stripe(r0, nrows):
        for t in range(nrows // ZR):
            _fill_iota(zidx_v, ZR, r0 + t * ZR)
            pltpu.sync_copy(zbuf, s_agg.at[zidx_v])

    _per_stripe(sid, _zero_stripe)

    @pl.when(sid == 0)
    def _():
        _fill_iota(zidx_v, ZR, 0)
        pltpu.sync_copy(zbuf, s_degp.at[zidx_v])

    plsc.subcore_barrier()

    def ebody(i, c):
        base = pl.multiple_of(wid * EPW + i * EK, 8)
        pltpu.sync_copy(src_ref.at[pl.ds(base, EK)], src_v)
        pltpu.sync_copy(dst_ref.at[pl.ds(base, EK)], dst_v)
        pltpu.async_copy(x_ref.at[src_v], rows_v, sem).wait()
        pltpu.sync_copy(rows_v, s_agg.at[dst_v], add=True)

        for off, _ in _deg_groups():
            dv = dst_v[pl.ds(off, 16)]
            prow_v[pl.ds(off, 16)] = lax.shift_right_logical(dv, 7)
            pcol_v[pl.ds(off, 16)] = lax.bitwise_and(dv, 127)
        pltpu.async_copy(eye_ref.at[pcol_v], hot_v, sem).wait()
        pltpu.sync_copy(hot_v, s_degp.at[prow_v], add=True)
        return c

    lax.fori_loop(0, NCHUNK, ebody, 0)
    plsc.subcore_barrier()

    def _out_stripe(r0, nrows):
        for t in range(nrows // ZR):
            r = r0 + t * ZR
            _fill_iota(zidx_v, ZR, r)
            pltpu.async_copy(s_agg.at[zidx_v], zbuf, sem).wait()
            pltpu.sync_copy(zbuf, agg.at[cid, pl.ds(r, ZR)])

    _per_stripe(sid, _out_stripe)

    @pl.when(sid == 0)
    def _():
        _fill_iota(zidx_v, ZR, 0)
        pltpu.async_copy(s_degp.at[zidx_v], zbuf, sem).wait()
        pltpu.sync_copy(zbuf, degp_out.at[cid])


def _sc_layer2_body(src_ref, dst_ref, hr_ref, tgt_ref,
                    att, hrt,
                    src_v, dst_v, rows_v, tix_v, tix2_v, tbuf, zbuf, zidx_v,
                    s_agg, sem):
    cid = lax.axis_index("c")
    sid = lax.axis_index("s")
    wid = sid * NC + cid

    _fill_rows(zbuf, ZR, D // 16, jnp.zeros((16,), jnp.float32))

    def _zero_stripe(r0, nrows):
        for t in range(nrows // ZR):
            _fill_iota(zidx_v, ZR, r0 + t * ZR)
            pltpu.sync_copy(zbuf, s_agg.at[zidx_v])

    _per_stripe(sid, _zero_stripe)
    plsc.subcore_barrier()

    def ebody(i, c):
        base = pl.multiple_of(wid * EPW + i * EK, 8)
        pltpu.sync_copy(src_ref.at[pl.ds(base, EK)], src_v)
        pltpu.sync_copy(dst_ref.at[pl.ds(base, EK)], dst_v)
        pltpu.async_copy(hr_ref.at[src_v], rows_v, sem).wait()
        pltpu.sync_copy(rows_v, s_agg.at[dst_v], add=True)
        return c

    lax.fori_loop(0, NCHUNK, ebody, 0)
    plsc.subcore_barrier()

    t0 = sid * TPS
    pltpu.sync_copy(tgt_ref.at[pl.ds(t0, TPS)], tix_v)
    pltpu.async_copy(s_agg.at[tix_v], tbuf, sem).wait()
    pltpu.sync_copy(tbuf, att.at[cid, pl.ds(t0, TPS)])

    off = cid * N_NODES
    for j in range(TPS // 16):
        tix2_v[pl.ds(j * 16, 16)] = tix_v[pl.ds(j * 16, 16)] + off
    pltpu.async_copy(hr_ref.at[tix2_v], tbuf, sem).wait()
    pltpu.sync_copy(tbuf, hrt.at[cid, pl.ds(t0, TPS)])


def _sc_layer1(src, dst, x):
    f32 = jnp.float32
    out_type = (
        jax.ShapeDtypeStruct((NC, N_NODES, D), f32),
        jax.ShapeDtypeStruct((NC, DEGP_R, D), f32),
    )
    scratch = [
        pltpu.VMEM((EK,), jnp.int32),
        pltpu.VMEM((EK,), jnp.int32),
        pltpu.VMEM((EK, D), f32),
        pltpu.VMEM((EK, D), f32),
        pltpu.VMEM((EK,), jnp.int32),
        pltpu.VMEM((EK,), jnp.int32),
        pltpu.VMEM((ZR, D), f32),
        pltpu.VMEM((ZR,), jnp.int32),
        pltpu.VMEM_SHARED((N_NODES, D), f32),
        pltpu.VMEM_SHARED((DEGP_R, D), f32),
        pltpu.SemaphoreType.DMA,
    ]
    k = pl.kernel(_sc_layer1_body, out_type=out_type, mesh=_mesh(),
                  scratch_types=scratch)
    return k(src, dst, x, jnp.eye(D, dtype=f32))


def _sc_layer2(src, dst, hr, tgt):
    f32 = jnp.float32
    out_type = (
        jax.ShapeDtypeStruct((NC, NT, D), f32),
        jax.ShapeDtypeStruct((NC, NT, D), f32),
    )
    scratch = [
        pltpu.VMEM((EK,), jnp.int32),
        pltpu.VMEM((EK,), jnp.int32),
        pltpu.VMEM((EK, D), f32),
        pltpu.VMEM((TPS,), jnp.int32),
        pltpu.VMEM((TPS,), jnp.int32),
        pltpu.VMEM((TPS, D), f32),
        pltpu.VMEM((ZR, D), f32),
        pltpu.VMEM((ZR,), jnp.int32),
        pltpu.VMEM_SHARED((N_NODES, D), f32),
        pltpu.SemaphoreType.DMA,
    ]
    k = pl.kernel(_sc_layer2_body, out_type=out_type, mesh=_mesh(),
                  scratch_types=scratch)
    return k(src, dst, hr, tgt)


def _tc1_body(x_ref, a_ref, d_ref, ws, wn, b, o_ref, rd_ref):
    deg = d_ref[...]
    rdeg = 1.0 / jnp.maximum(deg, 1.0)
    mean = (a_ref[0] + a_ref[1]) * rdeg
    h = (jnp.dot(x_ref[...], ws[...], preferred_element_type=jnp.float32)
         + jnp.dot(mean, wn[...], preferred_element_type=jnp.float32)
         + b[...])
    o_ref[...] = jnp.maximum(h, 0.0)
    rd_ref[...] = jnp.broadcast_to(rdeg, rd_ref.shape)


def _tc1(x, agg, deg_col, ws, wn, b):
    BR = 1000
    grid = (N_NODES // BR,)
    return pl.pallas_call(
        _tc1_body,
        grid=grid,
        in_specs=[
            pl.BlockSpec((BR, D), lambda i: (i, 0)),
            pl.BlockSpec((NC, BR, D), lambda i: (0, i, 0)),
            pl.BlockSpec((BR, 1), lambda i: (i, 0)),
            pl.BlockSpec((D, D), lambda i: (0, 0)),
            pl.BlockSpec((D, D), lambda i: (0, 0)),
            pl.BlockSpec((1, D), lambda i: (0, 0)),
        ],
        out_specs=[
            pl.BlockSpec((BR, D), lambda i: (i, 0)),
            pl.BlockSpec((BR, D), lambda i: (i, 0)),
        ],
        out_shape=[
            jax.ShapeDtypeStruct((N_NODES, D), jnp.float32),
            jax.ShapeDtypeStruct((N_NODES, D), jnp.float32),
        ],
    )(x, agg, deg_col, ws, wn, b.reshape(1, D))


def _tc2_body(att, hrt, ws, wn, b, o_ref):
    mean = (att[0] + att[1]) * hrt[1]
    o_ref[...] = (jnp.dot(hrt[0], ws[...], preferred_element_type=jnp.float32)
                  + jnp.dot(mean, wn[...], preferred_element_type=jnp.float32)
                  + b[...])


def _tc2(att, hrt, ws, wn, b):
    return pl.pallas_call(
        _tc2_body,
        grid=(1,),
        in_specs=[
            pl.BlockSpec((NC, NT, D), lambda i: (0, 0, 0)),
            pl.BlockSpec((NC, NT, D), lambda i: (0, 0, 0)),
            pl.BlockSpec((D, D), lambda i: (0, 0)),
            pl.BlockSpec((D, D), lambda i: (0, 0)),
            pl.BlockSpec((1, D), lambda i: (0, 0)),
        ],
        out_specs=pl.BlockSpec((NT, D), lambda i: (0, 0)),
        out_shape=jax.ShapeDtypeStruct((NT, D), jnp.float32),
    )(att, hrt, ws, wn, b.reshape(1, D))


@jax.jit
def kernel(x, edge_index, target_indices, W_self1, W_neigh1, b1,
           W_self2, W_neigh2, b2):
    edge = edge_index.astype(jnp.int32)
    src = edge[0]
    dst = edge[1]
    tgt = target_indices.astype(jnp.int32)
    agg, degp = _sc_layer1(src, dst, x)
    deg_col = (degp[0] + degp[1]).reshape(DEGP_R * D)[:N_NODES].reshape(
        N_NODES, 1)
    h1, rdeg = _tc1(x, agg, deg_col, W_self1, W_neigh1, b1)
    hr = jnp.concatenate([h1, rdeg], axis=0)
    att, hrt = _sc_layer2(src, dst, hr, tgt)
    return _tc2(att, hrt, W_self2, W_neigh2, b2)


# speedup vs baseline: 4.3701x; 1.5182x over previous
import functools

import jax
import jax.numpy as jnp
from jax import lax
from jax.experimental import pallas as pl
from jax.experimental.pallas import tpu as pltpu
from jax.experimental.pallas import tpu_sc as plsc

N_NODES = 10000
N_EDGES = 320000
D = 128
NT = 1024

NC, NS = 2, 16
NW = NC * NS
EPW = N_EDGES // NW
EK = 80
NCHUNK = EPW // EK
SPR = 640
SPR_LAST = N_NODES - (NS - 1) * SPR
ZR = 80
TPS = NT // NS
DEGP_R = 80


def _mesh():
    return plsc.VectorSubcoreMesh(core_axis_name="c", subcore_axis_name="s",
                                  num_cores=NC, num_subcores=NS)


def _per_stripe(sid, fn):
    @pl.when(sid < NS - 1)
    def _():
        fn(pl.multiple_of(sid * SPR, 8), SPR)

    @pl.when(sid == NS - 1)
    def _():
        fn((NS - 1) * SPR, SPR_LAST)


def _fill_rows(buf, nrows, ncol16, val16):
    def frow(i, c):
        for j in range(ncol16):
            buf[i, pl.ds(j * 16, 16)] = val16
        return c

    lax.fori_loop(0, nrows, frow, 0)


def _fill_iota(idx_ref, n, base):
    for j in range(n // 16):
        idx_ref[pl.ds(j * 16, 16)] = lax.iota(jnp.int32, 16) + (base + j * 16)


def _sc_layer1_body(src_ref, dst_ref, x_ref, eye_ref,
                    agg, degp_out,
                    src_v, dst_v, rows_v, hot_v, prow_v, pcol_v, zbuf, zidx_v,
                    s_agg, s_degp, sem):
    cid = lax.axis_index("c")
    sid = lax.axis_index("s")
    wid = sid * NC + cid

    z16 = jnp.zeros((16,), jnp.float32)
    _fill_rows(zbuf, ZR, D // 16, z16)

    def _zero_stripe(r0, nrows):
        for t in range(nrows // ZR):
            _fill_iota(zidx_v, ZR, r0 + t * ZR)
            pltpu.sync_copy(zbuf, s_agg.at[zidx_v])

    _per_stripe(sid, _zero_stripe)

    @pl.when(sid == 0)
    def _():
        _fill_iota(zidx_v, ZR, 0)
        pltpu.sync_copy(zbuf, s_degp.at[zidx_v])

    plsc.subcore_barrier()

    def ebody(i, c):
        base = pl.multiple_of(wid * EPW + i * EK, 8)
        pltpu.sync_copy(src_ref.at[pl.ds(base, EK)], src_v)
        pltpu.sync_copy(dst_ref.at[pl.ds(base, EK)], dst_v)
        pltpu.async_copy(x_ref.at[src_v], rows_v, sem).wait()
        pltpu.sync_copy(rows_v, s_agg.at[dst_v], add=True)

        for off in range(0, EK, 16):
            dv = dst_v[pl.ds(off, 16)]
            prow_v[pl.ds(off, 16)] = lax.shift_right_logical(dv, 7)
            pcol_v[pl.ds(off, 16)] = lax.bitwise_and(dv, 127)
        pltpu.async_copy(eye_ref.at[pcol_v], hot_v, sem).wait()
        pltpu.sync_copy(hot_v, s_degp.at[prow_v], add=True)
        return c

    lax.fori_loop(0, NCHUNK, ebody, 0)
    plsc.subcore_barrier()

    def _out_stripe(r0, nrows):
        for t in range(nrows // ZR):
            r = r0 + t * ZR
            _fill_iota(zidx_v, ZR, r)
            pltpu.async_copy(s_agg.at[zidx_v], zbuf, sem).wait()
            pltpu.sync_copy(zbuf, agg.at[cid, pl.ds(r, ZR)])

    _per_stripe(sid, _out_stripe)

    @pl.when(sid == 0)
    def _():
        _fill_iota(zidx_v, ZR, 0)
        pltpu.async_copy(s_degp.at[zidx_v], zbuf, sem).wait()
        pltpu.sync_copy(zbuf, degp_out.at[cid])


def _sc_layer2_body(src_ref, dst_ref, hr_ref, tgt_ref,
                    att, hrt,
                    src_v, dst_v, rows_v, tix_v, tix2_v, tbuf, zbuf, zidx_v,
                    s_agg, sem):
    cid = lax.axis_index("c")
    sid = lax.axis_index("s")
    wid = sid * NC + cid

    _fill_rows(zbuf, ZR, D // 16, jnp.zeros((16,), jnp.float32))

    def _zero_stripe(r0, nrows):
        for t in range(nrows // ZR):
            _fill_iota(zidx_v, ZR, r0 + t * ZR)
            pltpu.sync_copy(zbuf, s_agg.at[zidx_v])

    _per_stripe(sid, _zero_stripe)
    plsc.subcore_barrier()

    def ebody(i, c):
        base = pl.multiple_of(wid * EPW + i * EK, 8)
        pltpu.sync_copy(src_ref.at[pl.ds(base, EK)], src_v)
        pltpu.sync_copy(dst_ref.at[pl.ds(base, EK)], dst_v)
        pltpu.async_copy(hr_ref.at[src_v], rows_v, sem).wait()
        pltpu.sync_copy(rows_v, s_agg.at[dst_v], add=True)
        return c

    lax.fori_loop(0, NCHUNK, ebody, 0)
    plsc.subcore_barrier()

    t0 = sid * TPS
    pltpu.sync_copy(tgt_ref.at[pl.ds(t0, TPS)], tix_v)
    pltpu.async_copy(s_agg.at[tix_v], tbuf, sem).wait()
    pltpu.sync_copy(tbuf, att.at[cid, pl.ds(t0, TPS)])

    off = cid * N_NODES
    for j in range(TPS // 16):
        tix2_v[pl.ds(j * 16, 16)] = tix_v[pl.ds(j * 16, 16)] + off
    pltpu.async_copy(hr_ref.at[tix2_v], tbuf, sem).wait()
    pltpu.sync_copy(tbuf, hrt.at[cid, pl.ds(t0, TPS)])


def _sc_layer1(src, dst, x):
    f32 = jnp.float32
    out_type = (
        jax.ShapeDtypeStruct((NC, N_NODES, D), f32),
        jax.ShapeDtypeStruct((NC, DEGP_R, D), f32),
    )
    scratch = [
        pltpu.VMEM((EK,), jnp.int32),
        pltpu.VMEM((EK,), jnp.int32),
        pltpu.VMEM((EK, D), f32),
        pltpu.VMEM((EK, D), f32),
        pltpu.VMEM((EK,), jnp.int32),
        pltpu.VMEM((EK,), jnp.int32),
        pltpu.VMEM((ZR, D), f32),
        pltpu.VMEM((ZR,), jnp.int32),
        pltpu.VMEM_SHARED((N_NODES, D), f32),
        pltpu.VMEM_SHARED((DEGP_R, D), f32),
        pltpu.SemaphoreType.DMA,
    ]
    k = pl.kernel(_sc_layer1_body, out_type=out_type, mesh=_mesh(),
                  scratch_types=scratch)
    return k(src, dst, x, jnp.eye(D, dtype=f32))


def _sc_layer2(src, dst, hr, tgt):
    f32 = jnp.float32
    out_type = (
        jax.ShapeDtypeStruct((NC, NT, D), f32),
        jax.ShapeDtypeStruct((NC, NT, D), f32),
    )
    scratch = [
        pltpu.VMEM((EK,), jnp.int32),
        pltpu.VMEM((EK,), jnp.int32),
        pltpu.VMEM((EK, D), f32),
        pltpu.VMEM((TPS,), jnp.int32),
        pltpu.VMEM((TPS,), jnp.int32),
        pltpu.VMEM((TPS, D), f32),
        pltpu.VMEM((ZR, D), f32),
        pltpu.VMEM((ZR,), jnp.int32),
        pltpu.VMEM_SHARED((N_NODES, D), f32),
        pltpu.SemaphoreType.DMA,
    ]
    k = pl.kernel(_sc_layer2_body, out_type=out_type, mesh=_mesh(),
                  scratch_types=scratch)
    return k(src, dst, hr, tgt)


def _tc1_body(x_ref, a_ref, d_ref, ws, wn, b, o_ref, rd_ref):
    deg = d_ref[...]
    rdeg = 1.0 / jnp.maximum(deg, 1.0)
    mean = (a_ref[0] + a_ref[1]) * rdeg
    h = (jnp.dot(x_ref[...], ws[...], preferred_element_type=jnp.float32)
         + jnp.dot(mean, wn[...], preferred_element_type=jnp.float32)
         + b[...])
    o_ref[...] = jnp.maximum(h, 0.0)
    rd_ref[...] = jnp.broadcast_to(rdeg, rd_ref.shape)


def _tc1(x, agg, deg_col, ws, wn, b):
    BR = 1000
    grid = (N_NODES // BR,)
    return pl.pallas_call(
        _tc1_body,
        grid=grid,
        in_specs=[
            pl.BlockSpec((BR, D), lambda i: (i, 0)),
            pl.BlockSpec((NC, BR, D), lambda i: (0, i, 0)),
            pl.BlockSpec((BR, 1), lambda i: (i, 0)),
            pl.BlockSpec((D, D), lambda i: (0, 0)),
            pl.BlockSpec((D, D), lambda i: (0, 0)),
            pl.BlockSpec((1, D), lambda i: (0, 0)),
        ],
        out_specs=[
            pl.BlockSpec((BR, D), lambda i: (i, 0)),
            pl.BlockSpec((BR, D), lambda i: (i, 0)),
        ],
        out_shape=[
            jax.ShapeDtypeStruct((N_NODES, D), jnp.float32),
            jax.ShapeDtypeStruct((N_NODES, D), jnp.float32),
        ],
    )(x, agg, deg_col, ws, wn, b.reshape(1, D))


def _tc2_body(att, hrt, ws, wn, b, o_ref):
    mean = (att[0] + att[1]) * hrt[1]
    o_ref[...] = (jnp.dot(hrt[0], ws[...], preferred_element_type=jnp.float32)
                  + jnp.dot(mean, wn[...], preferred_element_type=jnp.float32)
                  + b[...])


def _tc2(att, hrt, ws, wn, b):
    return pl.pallas_call(
        _tc2_body,
        grid=(1,),
        in_specs=[
            pl.BlockSpec((NC, NT, D), lambda i: (0, 0, 0)),
            pl.BlockSpec((NC, NT, D), lambda i: (0, 0, 0)),
            pl.BlockSpec((D, D), lambda i: (0, 0)),
            pl.BlockSpec((D, D), lambda i: (0, 0)),
            pl.BlockSpec((1, D), lambda i: (0, 0)),
        ],
        out_specs=pl.BlockSpec((NT, D), lambda i: (0, 0)),
        out_shape=jax.ShapeDtypeStruct((NT, D), jnp.float32),
    )(att, hrt, ws, wn, b.reshape(1, D))


@jax.jit
def kernel(x, edge_index, target_indices, W_self1, W_neigh1, b1,
           W_self2, W_neigh2, b2):
    edge = edge_index.astype(jnp.int32)
    src = edge[0]
    dst = edge[1]
    tgt = target_indices.astype(jnp.int32)
    agg, degp = _sc_layer1(src, dst, x)
    deg_col = (degp[0] + degp[1]).reshape(DEGP_R * D)[:N_NODES].reshape(
        N_NODES, 1)
    h1, rdeg = _tc1(x, agg, deg_col, W_self1, W_neigh1, b1)
    hr = jnp.concatenate([h1, rdeg], axis=0)
    att, hrt = _sc_layer2(src, dst, hr, tgt)
    return _tc2(att, hrt, W_self2, W_neigh2, b2)


# speedup vs baseline: 4.8064x; 1.0998x over previous
import functools

import jax
import jax.numpy as jnp
from jax import lax
from jax.experimental import pallas as pl
from jax.experimental.pallas import tpu as pltpu
from jax.experimental.pallas import tpu_sc as plsc

N_NODES = 10000
N_EDGES = 320000
D = 128
NT = 1024

NC, NS = 2, 16
NW = NC * NS
EPW = N_EDGES // NW
EK = 80
NCHUNK = EPW // EK
SPR = 640
SPR_LAST = N_NODES - (NS - 1) * SPR
ZR = 80
TPS = NT // NS
DEGP_R = 80


def _mesh():
    return plsc.VectorSubcoreMesh(core_axis_name="c", subcore_axis_name="s",
                                  num_cores=NC, num_subcores=NS)


def _per_stripe(sid, fn):
    @pl.when(sid < NS - 1)
    def _():
        fn(pl.multiple_of(sid * SPR, 8), SPR)

    @pl.when(sid == NS - 1)
    def _():
        fn((NS - 1) * SPR, SPR_LAST)


def _fill_rows(buf, nrows, ncol16, val16):
    def frow(i, c):
        for j in range(ncol16):
            buf[i, pl.ds(j * 16, 16)] = val16
        return c

    lax.fori_loop(0, nrows, frow, 0)


def _fill_iota(idx_ref, n, base):
    for j in range(n // 16):
        idx_ref[pl.ds(j * 16, 16)] = lax.iota(jnp.int32, 16) + (base + j * 16)


def _sc_layer1_body(edges_ref, x_ref, eye_ref,
                    agg, degp_out,
                    sd_v, rows_v, hot_v, prow_v, pcol_v, zbuf, zidx_v,
                    s_agg, s_degp, sem):
    cid = lax.axis_index("c")
    sid = lax.axis_index("s")
    wid = sid * NC + cid

    z16 = jnp.zeros((16,), jnp.float32)
    _fill_rows(zbuf, ZR, D // 16, z16)

    def _zero_stripe(r0, nrows):
        for t in range(nrows // ZR):
            _fill_iota(zidx_v, ZR, r0 + t * ZR)
            pltpu.sync_copy(zbuf, s_agg.at[zidx_v])

    _per_stripe(sid, _zero_stripe)

    @pl.when(sid == 0)
    def _():
        _fill_iota(zidx_v, ZR, 0)
        pltpu.sync_copy(zbuf, s_degp.at[zidx_v])

    plsc.subcore_barrier()

    def ebody(i, c):
        pltpu.sync_copy(edges_ref.at[wid * NCHUNK + i], sd_v)
        pltpu.async_copy(x_ref.at[sd_v.at[0]], rows_v, sem).wait()
        pltpu.sync_copy(rows_v, s_agg.at[sd_v.at[1]], add=True)

        for off in range(0, EK, 16):
            dv = sd_v[1, pl.ds(off, 16)]
            prow_v[pl.ds(off, 16)] = lax.shift_right_logical(dv, 7)
            pcol_v[pl.ds(off, 16)] = lax.bitwise_and(dv, 127)
        pltpu.async_copy(eye_ref.at[pcol_v], hot_v, sem).wait()
        pltpu.sync_copy(hot_v, s_degp.at[prow_v], add=True)
        return c

    lax.fori_loop(0, NCHUNK, ebody, 0)
    plsc.subcore_barrier()

    def _out_stripe(r0, nrows):
        for t in range(nrows // ZR):
            r = r0 + t * ZR
            _fill_iota(zidx_v, ZR, r)
            pltpu.async_copy(s_agg.at[zidx_v], zbuf, sem).wait()
            pltpu.sync_copy(zbuf, agg.at[cid, pl.ds(r, ZR)])

    _per_stripe(sid, _out_stripe)

    @pl.when(sid == 0)
    def _():
        _fill_iota(zidx_v, ZR, 0)
        pltpu.async_copy(s_degp.at[zidx_v], zbuf, sem).wait()
        pltpu.sync_copy(zbuf, degp_out.at[cid])


def _sc_layer2_body(edges_ref, hr_ref, tgt_ref,
                    att, hrt,
                    sd_v, rows_v, tix_v, tix2_v, tbuf, zbuf, zidx_v,
                    s_agg, sem):
    cid = lax.axis_index("c")
    sid = lax.axis_index("s")
    wid = sid * NC + cid

    _fill_rows(zbuf, ZR, D // 16, jnp.zeros((16,), jnp.float32))

    def _zero_stripe(r0, nrows):
        for t in range(nrows // ZR):
            _fill_iota(zidx_v, ZR, r0 + t * ZR)
            pltpu.sync_copy(zbuf, s_agg.at[zidx_v])

    _per_stripe(sid, _zero_stripe)
    plsc.subcore_barrier()

    def ebody(i, c):
        pltpu.sync_copy(edges_ref.at[wid * NCHUNK + i], sd_v)
        pltpu.async_copy(hr_ref.at[sd_v.at[0]], rows_v, sem).wait()
        pltpu.sync_copy(rows_v, s_agg.at[sd_v.at[1]], add=True)
        return c

    lax.fori_loop(0, NCHUNK, ebody, 0)
    plsc.subcore_barrier()

    t0 = sid * TPS
    pltpu.sync_copy(tgt_ref.at[pl.ds(t0, TPS)], tix_v)
    pltpu.async_copy(s_agg.at[tix_v], tbuf, sem).wait()
    pltpu.sync_copy(tbuf, att.at[cid, pl.ds(t0, TPS)])

    off = cid * N_NODES
    for j in range(TPS // 16):
        tix2_v[pl.ds(j * 16, 16)] = tix_v[pl.ds(j * 16, 16)] + off
    pltpu.async_copy(hr_ref.at[tix2_v], tbuf, sem).wait()
    pltpu.sync_copy(tbuf, hrt.at[cid, pl.ds(t0, TPS)])


def _sc_layer1(edges, x):
    f32 = jnp.float32
    out_type = (
        jax.ShapeDtypeStruct((NC, N_NODES, D), f32),
        jax.ShapeDtypeStruct((NC, DEGP_R, D), f32),
    )
    scratch = [
        pltpu.VMEM((2, EK), jnp.int32),
        pltpu.VMEM((EK, D), f32),
        pltpu.VMEM((EK, D), f32),
        pltpu.VMEM((EK,), jnp.int32),
        pltpu.VMEM((EK,), jnp.int32),
        pltpu.VMEM((ZR, D), f32),
        pltpu.VMEM((ZR,), jnp.int32),
        pltpu.VMEM_SHARED((N_NODES, D), f32),
        pltpu.VMEM_SHARED((DEGP_R, D), f32),
        pltpu.SemaphoreType.DMA,
    ]
    k = pl.kernel(_sc_layer1_body, out_type=out_type, mesh=_mesh(),
                  scratch_types=scratch)
    return k(edges, x, jnp.eye(D, dtype=f32))


def _sc_layer2(edges, hr, tgt):
    f32 = jnp.float32
    out_type = (
        jax.ShapeDtypeStruct((NC, NT, D), f32),
        jax.ShapeDtypeStruct((NC, NT, D), f32),
    )
    scratch = [
        pltpu.VMEM((2, EK), jnp.int32),
        pltpu.VMEM((EK, D), f32),
        pltpu.VMEM((TPS,), jnp.int32),
        pltpu.VMEM((TPS,), jnp.int32),
        pltpu.VMEM((TPS, D), f32),
        pltpu.VMEM((ZR, D), f32),
        pltpu.VMEM((ZR,), jnp.int32),
        pltpu.VMEM_SHARED((N_NODES, D), f32),
        pltpu.SemaphoreType.DMA,
    ]
    k = pl.kernel(_sc_layer2_body, out_type=out_type, mesh=_mesh(),
                  scratch_types=scratch)
    return k(edges, hr, tgt)


def _tc1_body(x_ref, a_ref, d_ref, ws, wn, b, o_ref, rd_ref):
    deg = d_ref[...]
    rdeg = 1.0 / jnp.maximum(deg, 1.0)
    mean = (a_ref[0] + a_ref[1]) * rdeg
    h = (jnp.dot(x_ref[...], ws[...], preferred_element_type=jnp.float32)
         + jnp.dot(mean, wn[...], preferred_element_type=jnp.float32)
         + b[...])
    o_ref[...] = jnp.maximum(h, 0.0)
    rd_ref[...] = jnp.broadcast_to(rdeg, rd_ref.shape)


def _tc1(x, agg, deg_col, ws, wn, b):
    BR = 1000
    grid = (N_NODES // BR,)
    return pl.pallas_call(
        _tc1_body,
        grid=grid,
        in_specs=[
            pl.BlockSpec((BR, D), lambda i: (i, 0)),
            pl.BlockSpec((NC, BR, D), lambda i: (0, i, 0)),
            pl.BlockSpec((BR, 1), lambda i: (i, 0)),
            pl.BlockSpec((D, D), lambda i: (0, 0)),
            pl.BlockSpec((D, D), lambda i: (0, 0)),
            pl.BlockSpec((1, D), lambda i: (0, 0)),
        ],
        out_specs=[
            pl.BlockSpec((BR, D), lambda i: (i, 0)),
            pl.BlockSpec((BR, D), lambda i: (i, 0)),
        ],
        out_shape=[
            jax.ShapeDtypeStruct((N_NODES, D), jnp.float32),
            jax.ShapeDtypeStruct((N_NODES, D), jnp.float32),
        ],
    )(x, agg, deg_col, ws, wn, b.reshape(1, D))


def _tc2_body(att, hrt, ws, wn, b, o_ref):
    mean = (att[0] + att[1]) * hrt[1]
    o_ref[...] = (jnp.dot(hrt[0], ws[...], preferred_element_type=jnp.float32)
                  + jnp.dot(mean, wn[...], preferred_element_type=jnp.float32)
                  + b[...])


def _tc2(att, hrt, ws, wn, b):
    return pl.pallas_call(
        _tc2_body,
        grid=(1,),
        in_specs=[
            pl.BlockSpec((NC, NT, D), lambda i: (0, 0, 0)),
            pl.BlockSpec((NC, NT, D), lambda i: (0, 0, 0)),
            pl.BlockSpec((D, D), lambda i: (0, 0)),
            pl.BlockSpec((D, D), lambda i: (0, 0)),
            pl.BlockSpec((1, D), lambda i: (0, 0)),
        ],
        out_specs=pl.BlockSpec((NT, D), lambda i: (0, 0)),
        out_shape=jax.ShapeDtypeStruct((NT, D), jnp.float32),
    )(att, hrt, ws, wn, b.reshape(1, D))


@jax.jit
def kernel(x, edge_index, target_indices, W_self1, W_neigh1, b1,
           W_self2, W_neigh2, b2):
    edge = edge_index.astype(jnp.int32)
    edges = jnp.stack(
        [edge[0].reshape(N_EDGES // EK, EK), edge[1].reshape(N_EDGES // EK, EK)],
        axis=1)
    tgt = target_indices.astype(jnp.int32)
    agg, degp = _sc_layer1(edges, x)
    deg_col = (degp[0] + degp[1]).reshape(DEGP_R * D)[:N_NODES].reshape(
        N_NODES, 1)
    h1, rdeg = _tc1(x, agg, deg_col, W_self1, W_neigh1, b1)
    hr = jnp.concatenate([h1, rdeg], axis=0)
    att, hrt = _sc_layer2(edges, hr, tgt)
    return _tc2(att, hrt, W_self2, W_neigh2, b2)


# speedup vs baseline: 5.5377x; 1.1522x over previous
import functools

import jax
import jax.numpy as jnp
from jax import lax
from jax.experimental import pallas as pl
from jax.experimental.pallas import tpu as pltpu
from jax.experimental.pallas import tpu_sc as plsc

N_NODES = 10000
N_EDGES = 320000
D = 128
NT = 1024

NC, NS = 2, 16
NW = NC * NS
EPW = N_EDGES // NW
EK = 80
NCHUNK = EPW // EK
SPR = 640
SPR_LAST = N_NODES - (NS - 1) * SPR
ZR = 80
TPS = NT // NS
DEGP_R = 80


def _mesh():
    return plsc.VectorSubcoreMesh(core_axis_name="c", subcore_axis_name="s",
                                  num_cores=NC, num_subcores=NS)


def _per_stripe(sid, fn):
    @pl.when(sid < NS - 1)
    def _():
        fn(pl.multiple_of(sid * SPR, 8), SPR)

    @pl.when(sid == NS - 1)
    def _():
        fn((NS - 1) * SPR, SPR_LAST)


def _fill_rows(buf, nrows, ncol16, val16):
    def frow(i, c):
        for j in range(ncol16):
            buf[i, pl.ds(j * 16, 16)] = val16
        return c

    lax.fori_loop(0, nrows, frow, 0)


def _fill_iota(idx_ref, n, base):
    for j in range(n // 16):
        idx_ref[pl.ds(j * 16, 16)] = lax.iota(jnp.int32, 16) + (base + j * 16)


def _sc_layer1_body(edges_ref, x_ref, eye_ref,
                    agg, degp_out,
                    sd_v, rows_v, hot_v, prow_v, pcol_v, zbuf, zidx_v,
                    s_agg, s_degp, sem, sem2):
    cid = lax.axis_index("c")
    sid = lax.axis_index("s")
    wid = sid * NC + cid

    z16 = jnp.zeros((16,), jnp.float32)
    _fill_rows(zbuf, ZR, D // 16, z16)

    def _zero_stripe(r0, nrows):
        for t in range(nrows // ZR):
            _fill_iota(zidx_v, ZR, r0 + t * ZR)
            pltpu.sync_copy(zbuf, s_agg.at[zidx_v])

    _per_stripe(sid, _zero_stripe)

    @pl.when(sid == 0)
    def _():
        _fill_iota(zidx_v, ZR, 0)
        pltpu.sync_copy(zbuf, s_degp.at[zidx_v])

    plsc.subcore_barrier()

    def ebody(i, c):
        pltpu.sync_copy(edges_ref.at[wid * NCHUNK + i], sd_v)
        gx = pltpu.async_copy(x_ref.at[sd_v.at[0]], rows_v, sem)
        for off in range(0, EK, 16):
            dv = sd_v[1, pl.ds(off, 16)]
            prow_v[pl.ds(off, 16)] = lax.shift_right_logical(dv, 7)
            pcol_v[pl.ds(off, 16)] = lax.bitwise_and(dv, 127)
        ge = pltpu.async_copy(eye_ref.at[pcol_v], hot_v, sem2)
        gx.wait()
        pltpu.sync_copy(rows_v, s_agg.at[sd_v.at[1]], add=True)
        ge.wait()
        pltpu.sync_copy(hot_v, s_degp.at[prow_v], add=True)
        return c

    lax.fori_loop(0, NCHUNK, ebody, 0)
    plsc.subcore_barrier()

    def _out_stripe(r0, nrows):
        for t in range(nrows // ZR):
            r = r0 + t * ZR
            _fill_iota(zidx_v, ZR, r)
            pltpu.async_copy(s_agg.at[zidx_v], zbuf, sem).wait()
            pltpu.sync_copy(zbuf, agg.at[cid, pl.ds(r, ZR)])

    _per_stripe(sid, _out_stripe)

    @pl.when(sid == 0)
    def _():
        _fill_iota(zidx_v, ZR, 0)
        pltpu.async_copy(s_degp.at[zidx_v], zbuf, sem).wait()
        pltpu.sync_copy(zbuf, degp_out.at[cid])


def _sc_layer2_body(edges_ref, hr_ref, tgt_ref,
                    att, hrt,
                    sd_v, rows_v, tix_v, tix2_v, tbuf, zbuf, zidx_v,
                    s_agg, sem):
    cid = lax.axis_index("c")
    sid = lax.axis_index("s")
    wid = sid * NC + cid

    _fill_rows(zbuf, ZR, D // 16, jnp.zeros((16,), jnp.float32))

    def _zero_stripe(r0, nrows):
        for t in range(nrows // ZR):
            _fill_iota(zidx_v, ZR, r0 + t * ZR)
            pltpu.sync_copy(zbuf, s_agg.at[zidx_v])

    _per_stripe(sid, _zero_stripe)
    plsc.subcore_barrier()

    def ebody(i, c):
        pltpu.sync_copy(edges_ref.at[wid * NCHUNK + i], sd_v)
        pltpu.async_copy(hr_ref.at[sd_v.at[0]], rows_v, sem).wait()
        pltpu.sync_copy(rows_v, s_agg.at[sd_v.at[1]], add=True)
        return c

    lax.fori_loop(0, NCHUNK, ebody, 0)
    plsc.subcore_barrier()

    t0 = sid * TPS
    pltpu.sync_copy(tgt_ref.at[pl.ds(t0, TPS)], tix_v)
    pltpu.async_copy(s_agg.at[tix_v], tbuf, sem).wait()
    pltpu.sync_copy(tbuf, att.at[cid, pl.ds(t0, TPS)])

    off = cid * N_NODES
    for j in range(TPS // 16):
        tix2_v[pl.ds(j * 16, 16)] = tix_v[pl.ds(j * 16, 16)] + off
    pltpu.async_copy(hr_ref.at[tix2_v], tbuf, sem).wait()
    pltpu.sync_copy(tbuf, hrt.at[cid, pl.ds(t0, TPS)])


def _sc_layer1(edges, x):
    f32 = jnp.float32
    out_type = (
        jax.ShapeDtypeStruct((NC, N_NODES, D), f32),
        jax.ShapeDtypeStruct((NC, DEGP_R, D), f32),
    )
    scratch = [
        pltpu.VMEM((2, EK), jnp.int32),
        pltpu.VMEM((EK, D), f32),
        pltpu.VMEM((EK, D), f32),
        pltpu.VMEM((EK,), jnp.int32),
        pltpu.VMEM((EK,), jnp.int32),
        pltpu.VMEM((ZR, D), f32),
        pltpu.VMEM((ZR,), jnp.int32),
        pltpu.VMEM_SHARED((N_NODES, D), f32),
        pltpu.VMEM_SHARED((DEGP_R, D), f32),
        pltpu.SemaphoreType.DMA,
        pltpu.SemaphoreType.DMA,
    ]
    k = pl.kernel(_sc_layer1_body, out_type=out_type, mesh=_mesh(),
                  scratch_types=scratch)
    return k(edges, x, jnp.eye(D, dtype=f32))


def _sc_layer2(edges, hr, tgt):
    f32 = jnp.float32
    out_type = (
        jax.ShapeDtypeStruct((NC, NT, D), f32),
        jax.ShapeDtypeStruct((NC, NT, D), f32),
    )
    scratch = [
        pltpu.VMEM((2, EK), jnp.int32),
        pltpu.VMEM((EK, D), f32),
        pltpu.VMEM((TPS,), jnp.int32),
        pltpu.VMEM((TPS,), jnp.int32),
        pltpu.VMEM((TPS, D), f32),
        pltpu.VMEM((ZR, D), f32),
        pltpu.VMEM((ZR,), jnp.int32),
        pltpu.VMEM_SHARED((N_NODES, D), f32),
        pltpu.SemaphoreType.DMA,
    ]
    k = pl.kernel(_sc_layer2_body, out_type=out_type, mesh=_mesh(),
                  scratch_types=scratch)
    return k(edges, hr, tgt)


def _tc1_body(x_ref, a_ref, d_ref, ws, wn, b, o_ref, rd_ref):
    deg = d_ref[...]
    rdeg = 1.0 / jnp.maximum(deg, 1.0)
    mean = (a_ref[0] + a_ref[1]) * rdeg
    h = (jnp.dot(x_ref[...], ws[...], preferred_element_type=jnp.float32)
         + jnp.dot(mean, wn[...], preferred_element_type=jnp.float32)
         + b[...])
    o_ref[...] = jnp.maximum(h, 0.0)
    rd_ref[...] = jnp.broadcast_to(rdeg, rd_ref.shape)


def _tc1(x, agg, deg_col, ws, wn, b):
    BR = 1000
    grid = (N_NODES // BR,)
    return pl.pallas_call(
        _tc1_body,
        grid=grid,
        in_specs=[
            pl.BlockSpec((BR, D), lambda i: (i, 0)),
            pl.BlockSpec((NC, BR, D), lambda i: (0, i, 0)),
            pl.BlockSpec((BR, 1), lambda i: (i, 0)),
            pl.BlockSpec((D, D), lambda i: (0, 0)),
            pl.BlockSpec((D, D), lambda i: (0, 0)),
            pl.BlockSpec((1, D), lambda i: (0, 0)),
        ],
        out_specs=[
            pl.BlockSpec((BR, D), lambda i: (i, 0)),
            pl.BlockSpec((BR, D), lambda i: (i, 0)),
        ],
        out_shape=[
            jax.ShapeDtypeStruct((N_NODES, D), jnp.float32),
            jax.ShapeDtypeStruct((N_NODES, D), jnp.float32),
        ],
    )(x, agg, deg_col, ws, wn, b.reshape(1, D))


def _tc2_body(att, hrt, ws, wn, b, o_ref):
    mean = (att[0] + att[1]) * hrt[1]
    o_ref[...] = (jnp.dot(hrt[0], ws[...], preferred_element_type=jnp.float32)
                  + jnp.dot(mean, wn[...], preferred_element_type=jnp.float32)
                  + b[...])


def _tc2(att, hrt, ws, wn, b):
    return pl.pallas_call(
        _tc2_body,
        grid=(1,),
        in_specs=[
            pl.BlockSpec((NC, NT, D), lambda i: (0, 0, 0)),
            pl.BlockSpec((NC, NT, D), lambda i: (0, 0, 0)),
            pl.BlockSpec((D, D), lambda i: (0, 0)),
            pl.BlockSpec((D, D), lambda i: (0, 0)),
            pl.BlockSpec((1, D), lambda i: (0, 0)),
        ],
        out_specs=pl.BlockSpec((NT, D), lambda i: (0, 0)),
        out_shape=jax.ShapeDtypeStruct((NT, D), jnp.float32),
    )(att, hrt, ws, wn, b.reshape(1, D))


@jax.jit
def kernel(x, edge_index, target_indices, W_self1, W_neigh1, b1,
           W_self2, W_neigh2, b2):
    edge = edge_index.astype(jnp.int32)
    edges = jnp.stack(
        [edge[0].reshape(N_EDGES // EK, EK), edge[1].reshape(N_EDGES // EK, EK)],
        axis=1)
    tgt = target_indices.astype(jnp.int32)
    agg, degp = _sc_layer1(edges, x)
    deg_col = (degp[0] + degp[1]).reshape(DEGP_R * D)[:N_NODES].reshape(
        N_NODES, 1)
    h1, rdeg = _tc1(x, agg, deg_col, W_self1, W_neigh1, b1)
    hr = jnp.concatenate([h1, rdeg], axis=0)
    att, hrt = _sc_layer2(edges, hr, tgt)
    return _tc2(att, hrt, W_self2, W_neigh2, b2)
